# baseline (speedup 1.0000x reference)
import functools
import math

import jax
import jax.numpy as jnp
from jax import lax
from jax.experimental import pallas as pl
from jax.experimental.pallas import tpu as pltpu
from jax.experimental.pallas import tpu_sc as plsc

_N = 10000
_E = 320000
_D = 128
_C = 64
_LAYERS = 4
_ALPHA = 0.5

_NC = 2
_NS = 16
_NW = _NC * _NS
_EPW = _E // _NW
_K = 104
_PAD = 296
_EPWP = _EPW + _PAD
_NCH = _EPWP // _K
_NP = 10240
_RPS = _NP // _NS
_ZR = 64
_TRASH = _NP - 1


def _segsum_body(x_hbm, src_hbm, dst_hbm, out_hbm, acc, sbufs, dbufs, rows,
                 gsem, ssem, dsem):
    cid = lax.axis_index("c")
    sid = lax.axis_index("s")
    wid = cid * _NS + sid

    zero = jnp.zeros((16,), jnp.float32)
    z0 = rows[0]

    def zstore(i, _):
        r = i // (_D // 16)
        c = i % (_D // 16)
        z0[r, pl.ds(c * 16, 16)] = zero
        return 0

    lax.fori_loop(0, _K * (_D // 16), zstore, 0)

    def zcopy(j, _):
        pltpu.sync_copy(z0, acc.at[pl.ds(sid * _RPS + j * _K, _K)])
        return 0

    lax.fori_loop(0, _RPS // _K, zcopy, 0)
    pltpu.sync_copy(z0.at[pl.ds(0, _RPS % _K)],
                    acc.at[pl.ds(sid * _RPS + (_RPS // _K) * _K, _RPS % _K)])

    ebase = wid * _EPWP

    def _sload(i, buf, p):
        return pltpu.make_async_copy(src_hbm.at[pl.ds(ebase + i * _K, _K)],
                                     buf, ssem.at[p])

    def _dload(i, buf, p):
        return pltpu.make_async_copy(dst_hbm.at[pl.ds(ebase + i * _K, _K)],
                                     buf, dsem.at[p])

    def _gather(i, p):
        return pltpu.make_async_copy(x_hbm.at[sbufs[p]], rows[p], gsem.at[p])

    for p in range(3):
        _sload(p, sbufs[p], p).start()
        _dload(p, dbufs[p], p).start()
    for p in range(2):
        _sload(p, sbufs[p], p).wait()
        _gather(p, p).start()
    plsc.subcore_barrier()

    def group(g, _):
        for p in range(3):
            i = g * 3 + p
            p2 = (p + 2) % 3
            _gather(i, p).wait()

            @pl.when(i + 2 < _NCH)
            def _():
                _sload(i + 2, sbufs[p2], p2).wait()
                _gather(i + 2, p2).start()

            _dload(i, dbufs[p], p).wait()
            pltpu.sync_copy(rows[p], acc.at[dbufs[p]], add=True)

            @pl.when(i + 3 < _NCH)
            def _():
                _sload(i + 3, sbufs[p], p).start()
                _dload(i + 3, dbufs[p], p).start()
        return 0

    lax.fori_loop(0, _NCH // 3, group, 0)
    plsc.subcore_barrier()
    pltpu.sync_copy(acc.at[pl.ds(sid * _RPS, _RPS)],
                    out_hbm.at[cid, pl.ds(sid * _RPS, _RPS)])


def _segsum(xcur, src, dst):
    mesh = plsc.VectorSubcoreMesh(core_axis_name="c", subcore_axis_name="s",
                                  num_cores=_NC, num_subcores=_NS)
    f = pl.kernel(
        _segsum_body,
        out_type=jax.ShapeDtypeStruct((_NC, _NP, _D), jnp.float32),
        mesh=mesh,
        scratch_types=[
            pltpu.VMEM_SHARED((_NP, _D), jnp.float32),
            [pltpu.VMEM((_K,), jnp.int32)] * 3,
            [pltpu.VMEM((_K,), jnp.int32)] * 3,
            [pltpu.VMEM((_K, _D), jnp.float32)] * 3,
            pltpu.SemaphoreType.DMA((3,)),
            pltpu.SemaphoreType.DMA((3,)),
            pltpu.SemaphoreType.DMA((3,)),
        ],
    )
    return f(xcur, src, dst)


_BR = 1000


def _init_tc(x, w0, b0):
    def body(x_ref, w_ref, b_ref, o_ref):
        h = jnp.dot(x_ref[...], w_ref[...],
                    preferred_element_type=jnp.float32) + b_ref[...]
        o_ref[...] = jnp.maximum(h, 0.0)

    return pl.pallas_call(
        body,
        grid=(_N // _BR,),
        in_specs=[pl.BlockSpec((_BR, _D), lambda i: (i, 0)),
                  pl.BlockSpec((_D, _D), lambda i: (0, 0)),
                  pl.BlockSpec((1, _D), lambda i: (0, 0))],
        out_specs=pl.BlockSpec((_BR, _D), lambda i: (i, 0)),
        out_shape=jax.ShapeDtypeStruct((_N, _D), jnp.float32),
    )(x, w0, b0.reshape(1, _D))


def _layer_tc(parts, x0, xcur, w, beta):
    def body(p_ref, x0_ref, xc_ref, w_ref, o_ref):
        agg = p_ref[0] + p_ref[1]
        t = (1.0 - _ALPHA) * agg + _ALPHA * x0_ref[...]
        out = (1.0 - beta) * t + beta * jnp.dot(
            t, w_ref[...], preferred_element_type=jnp.float32)
        o_ref[...] = jnp.maximum(out + xc_ref[...], 0.0)

    return pl.pallas_call(
        body,
        grid=(_N // _BR,),
        in_specs=[pl.BlockSpec((_NC, _BR, _D), lambda i: (0, i, 0)),
                  pl.BlockSpec((_BR, _D), lambda i: (i, 0)),
                  pl.BlockSpec((_BR, _D), lambda i: (i, 0)),
                  pl.BlockSpec((_D, _D), lambda i: (0, 0))],
        out_specs=pl.BlockSpec((_BR, _D), lambda i: (i, 0)),
        out_shape=jax.ShapeDtypeStruct((_N, _D), jnp.float32),
    )(parts, x0, xcur, w)


def _final_tc(xcur, w1, b1):
    def body(x_ref, w_ref, b_ref, o_ref):
        logits = jnp.dot(x_ref[...], w_ref[...],
                         preferred_element_type=jnp.float32) + b_ref[...]
        m = jnp.max(logits, axis=1, keepdims=True)
        z = logits - m
        lse = jnp.log(jnp.sum(jnp.exp(z), axis=1, keepdims=True))
        o_ref[...] = z - lse

    return pl.pallas_call(
        body,
        grid=(_N // _BR,),
        in_specs=[pl.BlockSpec((_BR, _D), lambda i: (i, 0)),
                  pl.BlockSpec((_D, _C), lambda i: (0, 0)),
                  pl.BlockSpec((1, _C), lambda i: (0, 0))],
        out_specs=pl.BlockSpec((_BR, _C), lambda i: (i, 0)),
        out_shape=jax.ShapeDtypeStruct((_N, _C), jnp.float32),
    )(xcur, w1, b1.reshape(1, _C))


def kernel(x, edge_index, w0, b0, conv_w, w1, b1):
    e = edge_index.reshape(2, _NW, _EPW)
    src = jnp.concatenate(
        [e[0], jnp.zeros((_NW, _PAD), jnp.int32)], axis=1).reshape(-1)
    dst = jnp.concatenate(
        [e[1], jnp.full((_NW, _PAD), _TRASH, jnp.int32)], axis=1).reshape(-1)
    h = _init_tc(x, w0, b0)
    x0 = h
    xcur = h
    for layer in range(_LAYERS):
        beta = math.log(1.0 / (layer + 1) + 1.0)
        parts = _segsum(xcur, src, dst)
        xcur = _layer_tc(parts, x0, xcur, conv_w[layer], beta)
    return _final_tc(xcur, w1, b1)


# speedup vs baseline: 2.3153x; 2.3153x over previous
import functools
import math

import jax
import jax.numpy as jnp
from jax import lax
from jax.experimental import pallas as pl
from jax.experimental.pallas import tpu as pltpu
from jax.experimental.pallas import tpu_sc as plsc

_N = 10000
_E = 320000
_D = 128
_C = 64
_LAYERS = 4
_ALPHA = 0.5

_NC = 2
_NS = 16
_NW = _NC * _NS
_EPW = _E // _NW
_K = 96
_PAD = 80
_EPWP = _EPW + _PAD
_NCH = _EPWP // _K
_NP = 10240
_RPS = _NP // _NS
_ZR = 64
_TRASH = _NP - 1


def _segsum_body(x_hbm, src_hbm, dst_hbm, out_hbm, acc, sbufs, dbufs, rows,
                 gsem, ssem, dsem, csem):
    cid = lax.axis_index("c")
    sid = lax.axis_index("s")
    wid = cid * _NS + sid

    zero = jnp.zeros((16,), jnp.float32)
    z0 = rows[0]

    def zstore(i, _):
        r = i // (_D // 16)
        c = i % (_D // 16)
        z0[r, pl.ds(c * 16, 16)] = zero
        return 0

    lax.fori_loop(0, _K * (_D // 16), zstore, 0)

    def zcopy(j, _):
        pltpu.sync_copy(z0, acc.at[pl.ds(sid * _RPS + j * _K, _K)])
        return 0

    lax.fori_loop(0, _RPS // _K, zcopy, 0)
    pltpu.sync_copy(z0.at[pl.ds(0, _RPS % _K)],
                    acc.at[pl.ds(sid * _RPS + (_RPS // _K) * _K, _RPS % _K)])

    ebase = wid * _EPWP

    def _sload(i, buf, p):
        return pltpu.make_async_copy(src_hbm.at[pl.ds(ebase + i * _K, _K)],
                                     buf, ssem.at[p])

    def _dload(i, buf, p):
        return pltpu.make_async_copy(dst_hbm.at[pl.ds(ebase + i * _K, _K)],
                                     buf, dsem.at[p])

    def _gather(i, p):
        return pltpu.make_async_copy(x_hbm.at[sbufs[p]], rows[p], gsem.at[p])

    def _scatter(p):
        return pltpu.make_async_copy(rows[p], acc.at[dbufs[p]], csem.at[p])

    for p in range(3):
        _sload(p, sbufs[p], p).start()
    for p in range(2):
        _dload(p, dbufs[p], p).start()
    for p in range(2):
        _sload(p, sbufs[p], p).wait()
        _gather(p, p).start()
    plsc.subcore_barrier()

    def group(g, _):
        for p in range(3):
            i = g * 3 + p
            p2 = (p + 2) % 3
            _gather(i, p).wait()

            @pl.when((i >= 1) & (i + 2 < _NCH))
            def _():
                _scatter(p2).wait()

            @pl.when(i + 2 < _NCH)
            def _():
                _dload(i + 2, dbufs[p2], p2).start()
                _sload(i + 2, sbufs[p2], p2).wait()
                _gather(i + 2, p2).start()

            _dload(i, dbufs[p], p).wait()
            _scatter(p).start(add=True)

            @pl.when(i + 3 < _NCH)
            def _():
                _sload(i + 3, sbufs[p], p).start()
        return 0

    lax.fori_loop(0, _NCH // 3, group, 0)
    for p in range(3):
        _scatter(p).wait()
    plsc.subcore_barrier()
    pltpu.sync_copy(acc.at[pl.ds(sid * _RPS, _RPS)],
                    out_hbm.at[cid, pl.ds(sid * _RPS, _RPS)])


def _segsum(xcur, src, dst):
    mesh = plsc.VectorSubcoreMesh(core_axis_name="c", subcore_axis_name="s",
                                  num_cores=_NC, num_subcores=_NS)
    f = pl.kernel(
        _segsum_body,
        out_type=jax.ShapeDtypeStruct((_NC, _NP, _D), jnp.float32),
        mesh=mesh,
        scratch_types=[
            pltpu.VMEM_SHARED((_NP, _D), jnp.float32),
            [pltpu.VMEM((_K,), jnp.int32)] * 3,
            [pltpu.VMEM((_K,), jnp.int32)] * 3,
            [pltpu.VMEM((_K, _D), jnp.float32)] * 3,
            pltpu.SemaphoreType.DMA((3,)),
            pltpu.SemaphoreType.DMA((3,)),
            pltpu.SemaphoreType.DMA((3,)),
            pltpu.SemaphoreType.DMA((3,)),
        ],
    )
    return f(xcur, src, dst)


_BR = 1000


def _init_tc(x, w0, b0):
    def body(x_ref, w_ref, b_ref, o_ref):
        h = jnp.dot(x_ref[...], w_ref[...],
                    preferred_element_type=jnp.float32) + b_ref[...]
        o_ref[...] = jnp.maximum(h, 0.0)

    return pl.pallas_call(
        body,
        grid=(_N // _BR,),
        in_specs=[pl.BlockSpec((_BR, _D), lambda i: (i, 0)),
                  pl.BlockSpec((_D, _D), lambda i: (0, 0)),
                  pl.BlockSpec((1, _D), lambda i: (0, 0))],
        out_specs=pl.BlockSpec((_BR, _D), lambda i: (i, 0)),
        out_shape=jax.ShapeDtypeStruct((_N, _D), jnp.float32),
    )(x, w0, b0.reshape(1, _D))


def _layer_tc(parts, x0, xcur, w, beta):
    def body(p_ref, x0_ref, xc_ref, w_ref, o_ref):
        agg = p_ref[0] + p_ref[1]
        t = (1.0 - _ALPHA) * agg + _ALPHA * x0_ref[...]
        out = (1.0 - beta) * t + beta * jnp.dot(
            t, w_ref[...], preferred_element_type=jnp.float32)
        o_ref[...] = jnp.maximum(out + xc_ref[...], 0.0)

    return pl.pallas_call(
        body,
        grid=(_N // _BR,),
        in_specs=[pl.BlockSpec((_NC, _BR, _D), lambda i: (0, i, 0)),
                  pl.BlockSpec((_BR, _D), lambda i: (i, 0)),
                  pl.BlockSpec((_BR, _D), lambda i: (i, 0)),
                  pl.BlockSpec((_D, _D), lambda i: (0, 0))],
        out_specs=pl.BlockSpec((_BR, _D), lambda i: (i, 0)),
        out_shape=jax.ShapeDtypeStruct((_N, _D), jnp.float32),
    )(parts, x0, xcur, w)


def _final_tc(xcur, w1, b1):
    def body(x_ref, w_ref, b_ref, o_ref):
        logits = jnp.dot(x_ref[...], w_ref[...],
                         preferred_element_type=jnp.float32) + b_ref[...]
        m = jnp.max(logits, axis=1, keepdims=True)
        z = logits - m
        lse = jnp.log(jnp.sum(jnp.exp(z), axis=1, keepdims=True))
        o_ref[...] = z - lse

    return pl.pallas_call(
        body,
        grid=(_N // _BR,),
        in_specs=[pl.BlockSpec((_BR, _D), lambda i: (i, 0)),
                  pl.BlockSpec((_D, _C), lambda i: (0, 0)),
                  pl.BlockSpec((1, _C), lambda i: (0, 0))],
        out_specs=pl.BlockSpec((_BR, _C), lambda i: (i, 0)),
        out_shape=jax.ShapeDtypeStruct((_N, _C), jnp.float32),
    )(xcur, w1, b1.reshape(1, _C))


def kernel(x, edge_index, w0, b0, conv_w, w1, b1):
    e = edge_index.reshape(2, _NW, _EPW)
    src = jnp.concatenate(
        [e[0], jnp.zeros((_NW, _PAD), jnp.int32)], axis=1).reshape(-1)
    dst = jnp.concatenate(
        [e[1], jnp.full((_NW, _PAD), _TRASH, jnp.int32)], axis=1).reshape(-1)
    h = _init_tc(x, w0, b0)
    x0 = h
    xcur = h
    for layer in range(_LAYERS):
        beta = math.log(1.0 / (layer + 1) + 1.0)
        parts = _segsum(xcur, src, dst)
        xcur = _layer_tc(parts, x0, xcur, conv_w[layer], beta)
    return _final_tc(xcur, w1, b1)
